# Initial kernel scaffold; baseline (speedup 1.0000x reference)
import jax
import jax.numpy as jnp
from jax.experimental import pallas as pl


def kernel(x, adj_t, W1l, b1, W1r, W2l, b2, W2r, W3l, b3, W3r):
    raise NotImplementedError("write your pallas kernel here")


# speedup vs baseline: 1.0837x; 1.0837x over previous
import jax
import jax.numpy as jnp
from jax import lax
from jax.experimental import pallas as pl
from jax.experimental.pallas import tpu as pltpu
import jax.experimental.pallas.tpu_sc as plsc

N = 10000
NT = 10240
E = 320000
HID = 256
OUT = 47
OUTP = 128

K = 128
EPAD = 323584
EW1 = EPAD // 32
CH1 = EW1 // K
EW2 = EPAD // 16
CH2 = EW2 // K

Q1 = 2560
A1 = 2688
Q2 = 5120
A2 = 5248

DEGR = 1280

BR = 512
GR = NT // BR

_HIGH = jax.lax.Precision.HIGHEST
_mesh = lambda: plsc.VectorSubcoreMesh(core_axis_name="c", subcore_axis_name="s")


def _pass_local_idx(dst_pad, q, npass):
  rel = dst_pad[None, :] - (jnp.arange(npass, dtype=jnp.int32) * q)[:, None]
  ok = (rel >= 0) & (rel < q)
  return jnp.where(ok, rel, q + (dst_pad[None, :] & 31))


def _seg_sum_edge_split(with_deg):
  def body(feat, src, dloc, zc, *rest):
    if with_deg:
      (ones8t, d8, dcol, parts, degp, acc_sh, deg_sh, sidx, didx,
       d8v, dcolv, rows, orow, stage, sem) = rest
    else:
      parts, acc_sh, sidx, didx, rows, stage, sem = rest
    c = lax.axis_index("c")
    s = lax.axis_index("s")
    base0 = (c * 16 + s) * EW1
    DRS = DEGR // 16

    if with_deg:
      pltpu.sync_copy(zc.at[pl.ds(s * DRS, DRS)], rows.at[pl.ds(0, DRS)])
      pltpu.sync_copy(rows.at[pl.ds(0, DRS)], deg_sh.at[pl.ds(s * DRS, DRS)])

    for p in range(4):
      r0 = s * (A1 // 16)
      pltpu.sync_copy(zc.at[pl.ds(r0, A1 // 16)], stage)
      pltpu.sync_copy(stage, acc_sh.at[pl.ds(r0, A1 // 16)])
      plsc.subcore_barrier()

      def step(t, carry, p=p):
        b = base0 + t * K
        pltpu.sync_copy(src.at[pl.ds(b, K)], sidx)
        pltpu.sync_copy(dloc.at[p, pl.ds(b, K)], didx)
        pltpu.async_copy(feat.at[sidx], rows, sem).wait()
        pltpu.sync_copy(rows, acc_sh.at[didx], add=True)
        if with_deg and p == 0:
          pltpu.sync_copy(dcol.at[pl.ds(b, K)], dcolv)
          pltpu.sync_copy(d8.at[pl.ds(b, K)], d8v)
          pltpu.async_copy(ones8t.at[dcolv], orow, sem).wait()
          pltpu.sync_copy(orow, deg_sh.at[d8v], add=True)
        return carry

      lax.fori_loop(0, CH1, step, 0)
      plsc.subcore_barrier()
      pltpu.sync_copy(acc_sh.at[pl.ds(r0, A1 // 16)], stage)
      pltpu.sync_copy(stage, parts.at[c, p, pl.ds(r0, A1 // 16)])
      if with_deg and p == 0:
        pltpu.sync_copy(deg_sh.at[pl.ds(s * DRS, DRS)], orow.at[pl.ds(0, DRS)])
        pltpu.sync_copy(orow.at[pl.ds(0, DRS)], degp.at[c, pl.ds(s * DRS, DRS)])
      plsc.subcore_barrier()

  if with_deg:
    out_type = [jax.ShapeDtypeStruct((2, 4, A1, 128), jnp.float32),
                jax.ShapeDtypeStruct((2, DEGR, 128), jnp.float32)]
    scratch = [
        pltpu.VMEM_SHARED((A1, 128), jnp.float32),
        pltpu.VMEM_SHARED((DEGR, 128), jnp.float32),
        pltpu.VMEM((K,), jnp.int32),
        pltpu.VMEM((K,), jnp.int32),
        pltpu.VMEM((K,), jnp.int32),
        pltpu.VMEM((K,), jnp.int32),
        pltpu.VMEM((K, 128), jnp.float32),
        pltpu.VMEM((K, 128), jnp.float32),
        pltpu.VMEM((A1 // 16, 128), jnp.float32),
        pltpu.SemaphoreType.DMA,
    ]
  else:
    out_type = [jax.ShapeDtypeStruct((2, 4, A1, 128), jnp.float32)]
    scratch = [
        pltpu.VMEM_SHARED((A1, 128), jnp.float32),
        pltpu.VMEM((K,), jnp.int32),
        pltpu.VMEM((K,), jnp.int32),
        pltpu.VMEM((K, 128), jnp.float32),
        pltpu.VMEM((A1 // 16, 128), jnp.float32),
        pltpu.SemaphoreType.DMA,
    ]
  return pl.kernel(body, out_type=out_type, mesh=_mesh(),
                   scratch_types=scratch)


def _seg_sum_channel_split():
  def body(feat, src2, dlocd, zc, acc2, acc_sh, sidx, didx, rows,
           stage, sem):
    c = lax.axis_index("c")
    s = lax.axis_index("s")
    base0 = s * EW2
    r0 = s * (A2 // 16)

    for p in range(2):
      pltpu.sync_copy(zc.at[pl.ds(r0, A2 // 16)], stage)
      pltpu.sync_copy(stage, acc_sh.at[pl.ds(r0, A2 // 16)])
      plsc.subcore_barrier()

      def step(t, carry, p=p):
        b = base0 + t * K
        pltpu.sync_copy(src2.at[c, pl.ds(b, K)], sidx)
        pltpu.sync_copy(dlocd.at[p, pl.ds(b, K)], didx)
        pltpu.async_copy(feat.at[sidx], rows, sem).wait()
        pltpu.sync_copy(rows, acc_sh.at[didx], add=True)
        return carry

      lax.fori_loop(0, CH2, step, 0)
      plsc.subcore_barrier()
      pltpu.sync_copy(acc_sh.at[pl.ds(r0, A2 // 16)], stage)
      pltpu.sync_copy(stage, acc2.at[c, p, pl.ds(r0, A2 // 16)])
      plsc.subcore_barrier()

  return pl.kernel(
      body,
      out_type=[jax.ShapeDtypeStruct((2, 2, A2, 128), jnp.float32)],
      mesh=_mesh(),
      scratch_types=[
          pltpu.VMEM_SHARED((A2, 128), jnp.float32),
          pltpu.VMEM((K,), jnp.int32),
          pltpu.VMEM((K,), jnp.int32),
          pltpu.VMEM((K, 128), jnp.float32),
          pltpu.VMEM((A2 // 16, 128), jnp.float32),
          pltpu.SemaphoreType.DMA,
      ])


def _inv_deg(degv_ref):
  return 1.0 / jnp.maximum(degv_ref[...], 1.0)


def _agg4(parts_ref):
  return parts_ref[0, 0] + parts_ref[1, 0]


def _tc1_body(parts_ref, degp_ref, x_ref, wl_ref, b_ref, wr_ref, out_ref):
  agg = _agg4(parts_ref) * _inv_deg(degp_ref)
  h = (jnp.dot(agg, wl_ref[...], precision=_HIGH)
       + b_ref[...][None, :]
       + jnp.dot(x_ref[...], wr_ref[...], precision=_HIGH))
  out_ref[0] = jnp.maximum(h, 0.0)


def _layer1_tc(parts, degp, x, wl, b, wr):
  return pl.pallas_call(
      _tc1_body,
      grid=(GR, 2),
      in_specs=[
          pl.BlockSpec((2, 1, BR, 128), lambda i, c: (0, i // 5, i % 5, 0)),
          pl.BlockSpec((BR, 1), lambda i, c: (i, 0)),
          pl.BlockSpec((BR, 128), lambda i, c: (i, 0)),
          pl.BlockSpec((128, 128), lambda i, c: (0, c)),
          pl.BlockSpec((128,), lambda i, c: (c,)),
          pl.BlockSpec((128, 128), lambda i, c: (0, c)),
      ],
      out_specs=pl.BlockSpec((1, BR, 128), lambda i, c: (c, i, 0)),
      out_shape=jax.ShapeDtypeStruct((2, NT, 128), jnp.float32),
  )(parts, degp, x, wl, b, wr)


def _tc2_body(acc2_ref, degp_ref, h1_ref, wl_ref, b_ref, wr_ref, w3lp_ref,
              h2_ref, p_ref):
  agg = jnp.concatenate([acc2_ref[0, 0], acc2_ref[1, 0]], axis=1)
  agg = agg * _inv_deg(degp_ref)
  h1f = jnp.concatenate([h1_ref[0], h1_ref[1]], axis=1)
  h2 = (jnp.dot(agg, wl_ref[...], precision=_HIGH)
        + b_ref[...][None, :]
        + jnp.dot(h1f, wr_ref[...], precision=_HIGH))
  h2 = jnp.maximum(h2, 0.0)
  h2_ref[...] = h2
  p_ref[...] = jnp.dot(h2, w3lp_ref[...], precision=_HIGH)


def _layer2_tc(acc2, degp, h1, wl, b, wr, w3lp):
  return pl.pallas_call(
      _tc2_body,
      grid=(GR,),
      in_specs=[
          pl.BlockSpec((2, 1, BR, 128), lambda i: (0, i // 10, i % 10, 0)),
          pl.BlockSpec((BR, 1), lambda i: (i, 0)),
          pl.BlockSpec((2, BR, 128), lambda i: (0, i, 0)),
          pl.BlockSpec((HID, HID), lambda i: (0, 0)),
          pl.BlockSpec((HID,), lambda i: (0,)),
          pl.BlockSpec((HID, HID), lambda i: (0, 0)),
          pl.BlockSpec((HID, OUTP), lambda i: (0, 0)),
      ],
      out_specs=[
          pl.BlockSpec((BR, HID), lambda i: (i, 0)),
          pl.BlockSpec((BR, OUTP), lambda i: (i, 0)),
      ],
      out_shape=[
          jax.ShapeDtypeStruct((NT, HID), jnp.float32),
          jax.ShapeDtypeStruct((NT, OUTP), jnp.float32),
      ],
  )(acc2, degp, h1, wl, b, wr, w3lp)


def _tc3_body(parts_ref, degp_ref, h2_ref, w3rp_ref, b3p_ref, out_ref):
  agg = _agg4(parts_ref) * _inv_deg(degp_ref)
  z = (agg + b3p_ref[...][None, :]
       + jnp.dot(h2_ref[...], w3rp_ref[...], precision=_HIGH))
  col = lax.broadcasted_iota(jnp.int32, z.shape, 1)
  valid = col < OUT
  m = jnp.max(jnp.where(valid, z, -jnp.inf), axis=1, keepdims=True)
  ex = jnp.where(valid, jnp.exp(z - m), 0.0)
  lse = jnp.log(jnp.sum(ex, axis=1, keepdims=True)) + m
  out_ref[...] = (z - lse)[:, :OUT]


def _layer3_tc(parts3, degp, h2, w3rp, b3p):
  return pl.pallas_call(
      _tc3_body,
      grid=(GR,),
      in_specs=[
          pl.BlockSpec((2, 1, BR, OUTP), lambda i: (0, i // 5, i % 5, 0)),
          pl.BlockSpec((BR, 1), lambda i: (i, 0)),
          pl.BlockSpec((BR, HID), lambda i: (i, 0)),
          pl.BlockSpec((HID, OUTP), lambda i: (0, 0)),
          pl.BlockSpec((OUTP,), lambda i: (0,)),
      ],
      out_specs=pl.BlockSpec((BR, OUT), lambda i: (i, 0)),
      out_shape=jax.ShapeDtypeStruct((NT, OUT), jnp.float32),
  )(parts3, degp, h2, w3rp, b3p)


def kernel(x, adj_t, W1l, b1, W1r, W2l, b2, W2r, W3l, b3, W3r):
  src = adj_t[0].astype(jnp.int32)
  dst = adj_t[1].astype(jnp.int32)
  pad = EPAD - E
  src_pad = jnp.concatenate([src, jnp.zeros((pad,), jnp.int32)])
  dst_pad = jnp.concatenate([dst, jnp.full((pad,), N, jnp.int32)])
  src2 = jnp.stack([src_pad, src_pad + NT])
  dloc1 = _pass_local_idx(dst_pad, Q1, 4)
  dlocd = _pass_local_idx(dst_pad, Q2, 2)

  xp = jnp.pad(x, ((0, NT - N), (0, 0)))
  zc = jnp.zeros((A2, 128), jnp.float32)
  oh = jnp.repeat(jnp.eye(8, dtype=jnp.float32), 16, axis=1)
  ones8t = jnp.concatenate([oh, jnp.zeros((8, 128), jnp.float32)], axis=0)
  d8 = jnp.right_shift(dst_pad, 3)
  dcol = jnp.bitwise_and(dst_pad, 7)

  w3lp = jnp.pad(W3l, ((0, 0), (0, OUTP - OUT)))
  w3rp = jnp.pad(W3r, ((0, 0), (0, OUTP - OUT)))
  b3p = jnp.pad(b3, (0, OUTP - OUT))

  parts1, degp = _seg_sum_edge_split(True)(
      xp, src_pad, dloc1, zc, ones8t, d8, dcol)
  degv = (degp[0] + degp[1]).reshape(NT, 16)[:, :1]
  h1 = _layer1_tc(parts1, degv, xp, W1l, b1, W1r)

  (acc2,) = _seg_sum_channel_split()(
      h1.reshape(2 * NT, 128), src2, dlocd, zc)
  h2, p = _layer2_tc(acc2, degv, h1, W2l, b2, W2r, w3lp)

  (parts3,) = _seg_sum_edge_split(False)(p, src_pad, dloc1, zc)
  return _layer3_tc(parts3, degv, h2, w3rp, b3p)[:N]
